# Initial kernel scaffold; baseline (speedup 1.0000x reference)
import jax
import jax.numpy as jnp
from jax.experimental import pallas as pl


def kernel(trigger_data, batched_chain, tx_start_time, batched_graphs, edge_index, W_gcn, b_gcn, W_t, b_t, W_n, b_n):
    raise NotImplementedError("write your pallas kernel here")


# speedup vs baseline: 290.6229x; 290.6229x over previous
import functools

import jax
import jax.numpy as jnp
from jax import lax
from jax.experimental import pallas as pl
from jax.experimental.pallas import tpu as pltpu
from jax.experimental.pallas import tpu_sc as plsc

N_NODES = 100000
N_EDGES = 6400000
NPAD = 102400
CHUNK = 2048
NCHUNKS = N_EDGES // CHUNK
NC, NS = 2, 16
NW = NC * NS
ITERS = -(-NCHUNKS // NW)
SUB = NPAD // NS
ZCH = 2048
NZ = NPAD // ZCH
KC = 4000
KN = N_NODES // KC

_sc_mesh = plsc.VectorSubcoreMesh(core_axis_name="c", subcore_axis_name="s")


def _fill_zero(zero_v):
    def body(i, _):
        zero_v[pl.ds(i * 16, 16)] = jnp.zeros((16,), jnp.float32)
        return 0
    lax.fori_loop(0, ZCH // 16, body, 0)


@functools.partial(
    pl.kernel,
    out_type=jax.ShapeDtypeStruct((NC * NPAD,), jnp.float32),
    mesh=_sc_mesh,
    scratch_types=[
        pltpu.VMEM((CHUNK,), jnp.int32),
        pltpu.VMEM((CHUNK,), jnp.float32),
        pltpu.VMEM((ZCH,), jnp.float32),
        pltpu.VMEM_SHARED((NPAD,), jnp.float32),
    ],
)
def _hist_kernel(dst_hbm, out_hbm, idx_v, ones_v, zero_v, cnt_sh):
    cid = lax.axis_index("c")
    sid = lax.axis_index("s")
    wid = sid * NC + cid

    _fill_zero(zero_v)

    def fill_ones(i, _):
        ones_v[pl.ds(i * 16, 16)] = jnp.full((16,), 1.0, jnp.float32)
        return 0

    lax.fori_loop(0, CHUNK // 16, fill_ones, 0)

    for i in range(-(-NZ // NS)):
        j = sid + NS * i

        @pl.when(j < NZ)
        def _():
            pltpu.sync_copy(zero_v, cnt_sh.at[pl.ds(j * ZCH, ZCH)])

    plsc.subcore_barrier()

    def chunk_body(i, _):
        k = wid + NW * i

        @pl.when(k < NCHUNKS)
        def _():
            pltpu.sync_copy(dst_hbm.at[pl.ds(k * CHUNK, CHUNK)], idx_v)
            pltpu.sync_copy(ones_v, cnt_sh.at[idx_v], add=True)

        return 0

    lax.fori_loop(0, ITERS, chunk_body, 0)
    plsc.subcore_barrier()
    pltpu.sync_copy(cnt_sh.at[pl.ds(sid * SUB, SUB)],
                    out_hbm.at[pl.ds(cid * NPAD + sid * SUB, SUB)])


@functools.partial(
    pl.kernel,
    out_type=jax.ShapeDtypeStruct((NC * NPAD,), jnp.float32),
    mesh=_sc_mesh,
    scratch_types=[
        pltpu.VMEM((CHUNK,), jnp.int32),
        pltpu.VMEM((CHUNK,), jnp.int32),
        pltpu.VMEM((CHUNK,), jnp.float32),
        pltpu.VMEM((ZCH,), jnp.float32),
        pltpu.VMEM_SHARED((NPAD,), jnp.float32),
        pltpu.VMEM_SHARED((NPAD,), jnp.float32),
    ],
)
def _msg_kernel(src_hbm, dst_hbm, d_hbm, out_hbm,
                si_v, di_v, val_v, zero_v, d_sh, s_sh):
    cid = lax.axis_index("c")
    sid = lax.axis_index("s")
    wid = sid * NC + cid

    _fill_zero(zero_v)
    pltpu.sync_copy(d_hbm.at[pl.ds(sid * SUB, SUB)], d_sh.at[pl.ds(sid * SUB, SUB)])

    @pl.when(cid == 0)
    def _():
        pltpu.sync_copy(d_hbm.at[pl.ds(sid * SUB, SUB)], s_sh.at[pl.ds(sid * SUB, SUB)])

    @pl.when(cid == 1)
    def _():
        for i in range(-(-NZ // NS)):
            j = sid + NS * i

            @pl.when(j < NZ)
            def _():
                pltpu.sync_copy(zero_v, s_sh.at[pl.ds(j * ZCH, ZCH)])

    plsc.subcore_barrier()

    def chunk_body(i, _):
        k = wid + NW * i

        @pl.when(k < NCHUNKS)
        def _():
            pltpu.sync_copy(src_hbm.at[pl.ds(k * CHUNK, CHUNK)], si_v)
            pltpu.sync_copy(dst_hbm.at[pl.ds(k * CHUNK, CHUNK)], di_v)
            pltpu.sync_copy(d_sh.at[si_v], val_v)
            pltpu.sync_copy(val_v, s_sh.at[di_v], add=True)

        return 0

    lax.fori_loop(0, ITERS, chunk_body, 0)
    plsc.subcore_barrier()
    pltpu.sync_copy(s_sh.at[pl.ds(sid * SUB, SUB)],
                    out_hbm.at[pl.ds(cid * NPAD + sid * SUB, SUB)])


def _rsqrt_body(c0_ref, c1_ref, d_ref):
    d_ref[...] = lax.rsqrt(c0_ref[...] + c1_ref[...] + 1.0)


_rsqrt_call = pl.pallas_call(
    _rsqrt_body,
    out_shape=jax.ShapeDtypeStruct((KN, 1, KC), jnp.float32),
    grid=(KN,),
    in_specs=[pl.BlockSpec((1, 1, KC), lambda i: (i, 0, 0))] * 2,
    out_specs=pl.BlockSpec((1, 1, KC), lambda i: (i, 0, 0)),
)


def _final_body(d_ref, s0_ref, s1_ref, wn_ref, wch_ref, wtr_ref, wtx_ref,
                trig_ref, chain_ref, tx_ref, wt_ref, bt_ref, bn_ref,
                wg_ref, bg_ref, out_ref, acc, colsum):
    i = pl.program_id(0)

    @pl.when(i == 0)
    def _():
        acc[...] = jnp.zeros_like(acc)
        colsum[...] = jnp.zeros_like(colsum)

    g = d_ref[0, 0, :] * (s0_ref[0, 0, :] + s1_ref[0, 0, :])
    w = wn_ref[...]
    acc[...] += jnp.dot(g.reshape(1, KC), w, preferred_element_type=jnp.float32)
    colsum[...] += jnp.sum(w, axis=0, keepdims=True)

    @pl.when(i == KN - 1)
    def _():
        v = wg_ref[...] * acc[...] + bg_ref[...] * colsum[...]
        trig = jnp.maximum(
            jnp.dot(trig_ref[...], wt_ref[...], preferred_element_type=jnp.float32)
            + bt_ref[...], 0.0)
        out_ref[...] = (
            v
            + chain_ref[...] * wch_ref[...]
            + jnp.dot(trig, wtr_ref[...], preferred_element_type=jnp.float32)
            + jnp.dot(tx_ref[...], wtx_ref[...], preferred_element_type=jnp.float32)
            + bn_ref[...]
        )


def _zero_map(*_):
    return None


_final_call = pl.pallas_call(
    _final_body,
    out_shape=jax.ShapeDtypeStruct((64, 128), jnp.float32),
    grid=(KN,),
    in_specs=[
        pl.BlockSpec((1, 1, KC), lambda i: (i, 0, 0)),
        pl.BlockSpec((1, 1, KC), lambda i: (i, 0, 0)),
        pl.BlockSpec((1, 1, KC), lambda i: (i, 0, 0)),
        pl.BlockSpec((KC, 128), lambda i: (i, 0)),
        pl.BlockSpec((1, 128), lambda i: (0, 0)),
        pl.BlockSpec((32, 128), lambda i: (0, 0)),
        pl.BlockSpec((8, 128), lambda i: (0, 0)),
        pl.BlockSpec((64, 16), lambda i: (0, 0)),
        pl.BlockSpec((64, 1), lambda i: (0, 0)),
        pl.BlockSpec((64, 8), lambda i: (0, 0)),
        pl.BlockSpec((16, 32), lambda i: (0, 0)),
        pl.BlockSpec((1, 32), lambda i: (0, 0)),
        pl.BlockSpec((1, 128), lambda i: (0, 0)),
        pl.BlockSpec((1, 1), lambda i: (0, 0)),
        pl.BlockSpec((1, 1), lambda i: (0, 0)),
    ],
    out_specs=pl.BlockSpec((64, 128), lambda i: (0, 0)),
    scratch_shapes=[
        pltpu.VMEM((1, 128), jnp.float32),
        pltpu.VMEM((1, 128), jnp.float32),
    ],
)


def kernel(trigger_data, batched_chain, tx_start_time, batched_graphs, edge_index,
           W_gcn, b_gcn, W_t, b_t, W_n, b_n):
    del batched_graphs
    src1d = edge_index[0]
    dst1d = edge_index[1]

    cnt = _hist_kernel(dst1d).reshape(NC, NPAD)
    c0 = cnt[0, :N_NODES].reshape(KN, 1, KC)
    c1 = cnt[1, :N_NODES].reshape(KN, 1, KC)
    d3 = _rsqrt_call(c0, c1)

    d_flat = jnp.concatenate(
        [d3.reshape(N_NODES), jnp.zeros((NPAD - N_NODES,), jnp.float32)])
    s = _msg_kernel(src1d, dst1d, d_flat).reshape(NC, NPAD)
    s0 = s[0, :N_NODES].reshape(KN, 1, KC)
    s1 = s[1, :N_NODES].reshape(KN, 1, KC)

    return _final_call(
        d3, s0, s1, W_n,
        W_n[N_NODES:N_NODES + 1],
        W_n[N_NODES + 1:N_NODES + 33],
        W_n[N_NODES + 33:N_NODES + 41],
        trigger_data,
        batched_chain.reshape(64, 1),
        tx_start_time,
        W_t,
        b_t.reshape(1, 32),
        b_n.reshape(1, 128),
        W_gcn,
        b_gcn.reshape(1, 1),
    )


# speedup vs baseline: 367.2617x; 1.2637x over previous
import functools

import jax
import jax.numpy as jnp
from jax import lax
from jax.experimental import pallas as pl
from jax.experimental.pallas import tpu as pltpu
from jax.experimental.pallas import tpu_sc as plsc

N_NODES = 100000
N_EDGES = 6400000
NPAD = 102400
CHUNK = 2048
NCHUNKS = N_EDGES // CHUNK
NC, NS = 2, 16
NW = NC * NS
NSETS = 4
ROUNDS = 25
SUB = NPAD // NS
ZCH = 2048
NZ = NPAD // ZCH
KC = 4000
KN = N_NODES // KC

_sc_mesh = plsc.VectorSubcoreMesh(core_axis_name="c", subcore_axis_name="s")


@functools.partial(
    pl.kernel,
    out_type=jax.ShapeDtypeStruct((NC * NPAD,), jnp.float32),
    mesh=_sc_mesh,
    scratch_types=[
        pltpu.VMEM((ZCH,), jnp.float32),
        pltpu.VMEM((CHUNK,), jnp.float32),
        pltpu.VMEM((CHUNK,), jnp.int32),
        pltpu.VMEM((CHUNK,), jnp.int32),
        pltpu.VMEM((CHUNK,), jnp.int32),
        pltpu.VMEM((CHUNK,), jnp.int32),
        pltpu.VMEM_SHARED((NPAD,), jnp.float32),
        pltpu.SemaphoreType.DMA,
        pltpu.SemaphoreType.DMA,
        pltpu.SemaphoreType.DMA,
        pltpu.SemaphoreType.DMA,
        pltpu.SemaphoreType.DMA,
        pltpu.SemaphoreType.DMA,
        pltpu.SemaphoreType.DMA,
        pltpu.SemaphoreType.DMA,
    ],
)
def _hist_kernel(dst_hbm, out_hbm, zero_v, ones_v, b0, b1, b2, b3, h_sh,
                 si0, si1, si2, si3, sc0, sc1, sc2, sc3):
    cid = lax.axis_index("c")
    sid = lax.axis_index("s")
    wid = sid * NC + cid
    bufs = (b0, b1, b2, b3)
    isems = (si0, si1, si2, si3)
    csems = (sc0, sc1, sc2, sc3)
    ones16 = jnp.full((16,), 1.0, jnp.float32)
    zeros16 = jnp.zeros((16,), jnp.float32)
    last = NCHUNKS - 1

    def zero_body(i, _):
        zero_v[pl.ds(i * 16, 16)] = zeros16
        return 0

    lax.fori_loop(0, ZCH // 16, zero_body, 0, unroll=8)

    def ones_body(i, _):
        ones_v[pl.ds(i * 16, 16)] = ones16
        return 0

    lax.fori_loop(0, CHUNK // 16, ones_body, 0, unroll=8)

    for i in range(-(-NZ // NS)):
        j = sid + NS * i

        @pl.when(j < NZ)
        def _(j=j):
            pltpu.sync_copy(zero_v, h_sh.at[pl.ds(j * ZCH, ZCH)])

    plsc.subcore_barrier()

    for b in range(NSETS):
        k = jnp.minimum(wid + NW * b, last)
        pltpu.async_copy(dst_hbm.at[pl.ds(k * CHUNK, CHUNK)], bufs[b], isems[b])

    def round_body(j, _):
        for b in range(NSETS):
            c = wid + NW * (NSETS * j + b)
            pltpu.make_async_copy(
                dst_hbm.at[pl.ds(0, CHUNK)], bufs[b], isems[b]).wait()

            @pl.when(c < NCHUNKS)
            def _(b=b):
                pltpu.async_copy(ones_v, h_sh.at[bufs[b]], csems[b], add=True)

        for b in range(NSETS):
            c = wid + NW * (NSETS * j + b)

            @pl.when(c < NCHUNKS)
            def _(b=b):
                pltpu.make_async_copy(ones_v, h_sh.at[bufs[b]], csems[b]).wait()

            k = jnp.minimum(c + NW * NSETS, last)
            pltpu.async_copy(dst_hbm.at[pl.ds(k * CHUNK, CHUNK)], bufs[b], isems[b])
        return 0

    lax.fori_loop(0, ROUNDS, round_body, 0)
    for b in range(NSETS):
        pltpu.make_async_copy(dst_hbm.at[pl.ds(0, CHUNK)], bufs[b], isems[b]).wait()
    plsc.subcore_barrier()

    for t in range(-(-NZ // NS)):
        j = sid + NS * t

        @pl.when(j < NZ)
        def _(j=j):
            pltpu.sync_copy(h_sh.at[pl.ds(j * ZCH, ZCH)], zero_v)
            pltpu.sync_copy(zero_v, out_hbm.at[pl.ds(cid * NPAD + j * ZCH, ZCH)])


@functools.partial(
    pl.kernel,
    out_type=jax.ShapeDtypeStruct((NC * NPAD,), jnp.float32),
    mesh=_sc_mesh,
    scratch_types=[
        pltpu.VMEM((ZCH,), jnp.float32),
        pltpu.VMEM((CHUNK,), jnp.int32),
        pltpu.VMEM((CHUNK,), jnp.int32),
        pltpu.VMEM((CHUNK,), jnp.int32),
        pltpu.VMEM((CHUNK,), jnp.int32),
        pltpu.VMEM((CHUNK,), jnp.int32),
        pltpu.VMEM((CHUNK,), jnp.int32),
        pltpu.VMEM((CHUNK,), jnp.int32),
        pltpu.VMEM((CHUNK,), jnp.int32),
        pltpu.VMEM((CHUNK,), jnp.float32),
        pltpu.VMEM((CHUNK,), jnp.float32),
        pltpu.VMEM((CHUNK,), jnp.float32),
        pltpu.VMEM((CHUNK,), jnp.float32),
        pltpu.VMEM_SHARED((NPAD,), jnp.float32),
        pltpu.VMEM_SHARED((NPAD,), jnp.float32),
        pltpu.SemaphoreType.DMA,
        pltpu.SemaphoreType.DMA,
        pltpu.SemaphoreType.DMA,
        pltpu.SemaphoreType.DMA,
        pltpu.SemaphoreType.DMA,
        pltpu.SemaphoreType.DMA,
        pltpu.SemaphoreType.DMA,
        pltpu.SemaphoreType.DMA,
        pltpu.SemaphoreType.DMA,
        pltpu.SemaphoreType.DMA,
        pltpu.SemaphoreType.DMA,
        pltpu.SemaphoreType.DMA,
        pltpu.SemaphoreType.DMA,
        pltpu.SemaphoreType.DMA,
        pltpu.SemaphoreType.DMA,
        pltpu.SemaphoreType.DMA,
    ],
)
def _msg_kernel(src_hbm, dst_hbm, d_hbm, out_hbm, zero_v,
                sb0, sb1, sb2, sb3, db0, db1, db2, db3, vb0, vb1, vb2, vb3,
                d_sh, s_sh,
                ss0, ss1, ss2, ss3, sd0, sd1, sd2, sd3,
                sg0, sg1, sg2, sg3, sc0, sc1, sc2, sc3):
    cid = lax.axis_index("c")
    sid = lax.axis_index("s")
    wid = sid * NC + cid
    sbufs = (sb0, sb1, sb2, sb3)
    dbufs = (db0, db1, db2, db3)
    vbufs = (vb0, vb1, vb2, vb3)
    ssems = (ss0, ss1, ss2, ss3)
    dsems = (sd0, sd1, sd2, sd3)
    gsems = (sg0, sg1, sg2, sg3)
    csems = (sc0, sc1, sc2, sc3)
    zeros16 = jnp.zeros((16,), jnp.float32)
    last = NCHUNKS - 1

    def zero_body(i, _):
        zero_v[pl.ds(i * 16, 16)] = zeros16
        return 0

    lax.fori_loop(0, ZCH // 16, zero_body, 0, unroll=8)

    for i in range(-(-NZ // NS)):
        j = sid + NS * i

        @pl.when(j < NZ)
        def _(j=j):
            pltpu.sync_copy(zero_v, s_sh.at[pl.ds(j * ZCH, ZCH)])

    for i in range(-(-NZ // NS)):
        j = sid + NS * i

        @pl.when(j < NZ)
        def _(j=j):
            pltpu.sync_copy(d_hbm.at[pl.ds(j * ZCH, ZCH)], vb0)
            pltpu.sync_copy(vb0, d_sh.at[pl.ds(j * ZCH, ZCH)])

    plsc.subcore_barrier()

    for b in range(NSETS):
        k = jnp.minimum(wid + NW * b, last)
        pltpu.async_copy(src_hbm.at[pl.ds(k * CHUNK, CHUNK)], sbufs[b], ssems[b])
        pltpu.async_copy(dst_hbm.at[pl.ds(k * CHUNK, CHUNK)], dbufs[b], dsems[b])

    def round_body(j, _):
        for b in range(NSETS):
            c = wid + NW * (NSETS * j + b)
            pltpu.make_async_copy(
                src_hbm.at[pl.ds(0, CHUNK)], sbufs[b], ssems[b]).wait()

            @pl.when(c < NCHUNKS)
            def _(b=b):
                pltpu.async_copy(d_sh.at[sbufs[b]], vbufs[b], gsems[b])

        for b in range(NSETS):
            c = wid + NW * (NSETS * j + b)
            pltpu.make_async_copy(
                dst_hbm.at[pl.ds(0, CHUNK)], dbufs[b], dsems[b]).wait()

            @pl.when(c < NCHUNKS)
            def _(b=b):
                pltpu.make_async_copy(d_sh.at[sbufs[b]], vbufs[b], gsems[b]).wait()
                pltpu.async_copy(vbufs[b], s_sh.at[dbufs[b]], csems[b], add=True)

        for b in range(NSETS):
            c = wid + NW * (NSETS * j + b)

            @pl.when(c < NCHUNKS)
            def _(b=b):
                pltpu.make_async_copy(vbufs[b], s_sh.at[dbufs[b]], csems[b]).wait()

            k = jnp.minimum(c + NW * NSETS, last)
            pltpu.async_copy(src_hbm.at[pl.ds(k * CHUNK, CHUNK)], sbufs[b], ssems[b])
            pltpu.async_copy(dst_hbm.at[pl.ds(k * CHUNK, CHUNK)], dbufs[b], dsems[b])
        return 0

    lax.fori_loop(0, ROUNDS, round_body, 0)
    for b in range(NSETS):
        pltpu.make_async_copy(src_hbm.at[pl.ds(0, CHUNK)], sbufs[b], ssems[b]).wait()
        pltpu.make_async_copy(dst_hbm.at[pl.ds(0, CHUNK)], dbufs[b], dsems[b]).wait()
    plsc.subcore_barrier()

    for t in range(-(-NZ // NS)):
        j = sid + NS * t

        @pl.when(j < NZ)
        def _(j=j):
            pltpu.sync_copy(s_sh.at[pl.ds(j * ZCH, ZCH)], zero_v)
            pltpu.sync_copy(zero_v, out_hbm.at[pl.ds(cid * NPAD + j * ZCH, ZCH)])


def _rsqrt_body(cnt_ref, d_ref):
    deg = cnt_ref[0, 0, 0, :] + cnt_ref[1, 0, 0, :]
    d_ref[0, 0, :] = lax.rsqrt(deg + 1.0)


_rsqrt_call = pl.pallas_call(
    _rsqrt_body,
    out_shape=jax.ShapeDtypeStruct((KN, 1, KC), jnp.float32),
    grid=(KN,),
    in_specs=[pl.BlockSpec((NC, 1, 1, KC), lambda i: (0, i, 0, 0))],
    out_specs=pl.BlockSpec((1, 1, KC), lambda i: (i, 0, 0)),
)


def _final_body(d_ref, s0_ref, s1_ref, wn_ref, wch_ref, wtr_ref, wtx_ref,
                trig_ref, chain_ref, tx_ref, wt_ref, bt_ref, bn_ref,
                wg_ref, bg_ref, out_ref, acc, colsum):
    i = pl.program_id(0)

    @pl.when(i == 0)
    def _():
        acc[...] = jnp.zeros_like(acc)
        colsum[...] = jnp.zeros_like(colsum)

    d = d_ref[0, 0, :]
    g = d * (s0_ref[0, 0, 0, :] + s1_ref[0, 0, 0, :] + d)
    w = wn_ref[...]
    acc[...] += jnp.dot(g.reshape(1, KC), w, preferred_element_type=jnp.float32)
    colsum[...] += jnp.sum(w, axis=0, keepdims=True)

    @pl.when(i == KN - 1)
    def _():
        v = wg_ref[...] * acc[...] + bg_ref[...] * colsum[...]
        trig = jnp.maximum(
            jnp.dot(trig_ref[...], wt_ref[...], preferred_element_type=jnp.float32)
            + bt_ref[...], 0.0)
        out_ref[...] = (
            v
            + chain_ref[...] * wch_ref[...]
            + jnp.dot(trig, wtr_ref[...], preferred_element_type=jnp.float32)
            + jnp.dot(tx_ref[...], wtx_ref[...], preferred_element_type=jnp.float32)
            + bn_ref[...]
        )


_final_call = pl.pallas_call(
    _final_body,
    out_shape=jax.ShapeDtypeStruct((64, 128), jnp.float32),
    grid=(KN,),
    in_specs=[
        pl.BlockSpec((1, 1, KC), lambda i: (i, 0, 0)),
        pl.BlockSpec((1, 1, 1, KC), lambda i: (0, i, 0, 0)),
        pl.BlockSpec((1, 1, 1, KC), lambda i: (1, i, 0, 0)),
        pl.BlockSpec((KC, 128), lambda i: (i, 0)),
        pl.BlockSpec((1, 128), lambda i: (0, 0)),
        pl.BlockSpec((32, 128), lambda i: (0, 0)),
        pl.BlockSpec((8, 128), lambda i: (0, 0)),
        pl.BlockSpec((64, 16), lambda i: (0, 0)),
        pl.BlockSpec((64, 1), lambda i: (0, 0)),
        pl.BlockSpec((64, 8), lambda i: (0, 0)),
        pl.BlockSpec((16, 32), lambda i: (0, 0)),
        pl.BlockSpec((1, 32), lambda i: (0, 0)),
        pl.BlockSpec((1, 128), lambda i: (0, 0)),
        pl.BlockSpec((1, 1), lambda i: (0, 0)),
        pl.BlockSpec((1, 1), lambda i: (0, 0)),
    ],
    out_specs=pl.BlockSpec((64, 128), lambda i: (0, 0)),
    scratch_shapes=[
        pltpu.VMEM((1, 128), jnp.float32),
        pltpu.VMEM((1, 128), jnp.float32),
    ],
)


def kernel(trigger_data, batched_chain, tx_start_time, batched_graphs, edge_index,
           W_gcn, b_gcn, W_t, b_t, W_n, b_n):
    del batched_graphs
    src1d = edge_index[0]
    dst1d = edge_index[1]

    cnt = _hist_kernel(dst1d).reshape(NC, NPAD)[:, :N_NODES]
    d3 = _rsqrt_call(cnt.reshape(NC, KN, 1, KC))

    d_flat = jnp.concatenate(
        [d3.reshape(N_NODES), jnp.zeros((NPAD - N_NODES,), jnp.float32)])
    sarr = _msg_kernel(src1d, dst1d, d_flat).reshape(NC, NPAD)[
        :, :N_NODES].reshape(NC, KN, 1, KC)

    return _final_call(
        d3, sarr, sarr, W_n,
        W_n[N_NODES:N_NODES + 1],
        W_n[N_NODES + 1:N_NODES + 33],
        W_n[N_NODES + 33:N_NODES + 41],
        trigger_data,
        batched_chain.reshape(64, 1),
        tx_start_time,
        W_t,
        b_t.reshape(1, 32),
        b_n.reshape(1, 128),
        W_gcn,
        b_gcn.reshape(1, 1),
    )


# speedup vs baseline: 370.7881x; 1.0096x over previous
import functools

import jax
import jax.numpy as jnp
from jax import lax
from jax.experimental import pallas as pl
from jax.experimental.pallas import tpu as pltpu
from jax.experimental.pallas import tpu_sc as plsc

N_NODES = 100000
N_EDGES = 6400000
NPAD = 104000
CHUNK = 2048
NCHUNKS = N_EDGES // CHUNK
NC, NS = 2, 16
NW = NC * NS
NSETS = 4
ROUNDS = 25
ZCH = 4000
NZ = NPAD // ZCH
KC = 4000
KN = N_NODES // KC

_sc_mesh = plsc.VectorSubcoreMesh(core_axis_name="c", subcore_axis_name="s")


@functools.partial(
    pl.kernel,
    out_type=jax.ShapeDtypeStruct((NC * NPAD,), jnp.float32),
    mesh=_sc_mesh,
    scratch_types=[
        pltpu.VMEM((ZCH,), jnp.float32),
        pltpu.VMEM((CHUNK,), jnp.float32),
        pltpu.VMEM((CHUNK,), jnp.int32),
        pltpu.VMEM((CHUNK,), jnp.int32),
        pltpu.VMEM((CHUNK,), jnp.int32),
        pltpu.VMEM((CHUNK,), jnp.int32),
        pltpu.VMEM_SHARED((NPAD,), jnp.float32),
        pltpu.SemaphoreType.DMA,
        pltpu.SemaphoreType.DMA,
        pltpu.SemaphoreType.DMA,
        pltpu.SemaphoreType.DMA,
        pltpu.SemaphoreType.DMA,
        pltpu.SemaphoreType.DMA,
        pltpu.SemaphoreType.DMA,
        pltpu.SemaphoreType.DMA,
    ],
)
def _hist_kernel(dst_hbm, out_hbm, zero_v, ones_v, b0, b1, b2, b3, h_sh,
                 si0, si1, si2, si3, sc0, sc1, sc2, sc3):
    cid = lax.axis_index("c")
    sid = lax.axis_index("s")
    wid = sid * NC + cid
    bufs = (b0, b1, b2, b3)
    isems = (si0, si1, si2, si3)
    csems = (sc0, sc1, sc2, sc3)
    ones16 = jnp.full((16,), 1.0, jnp.float32)
    zeros16 = jnp.zeros((16,), jnp.float32)
    last = NCHUNKS - 1

    def zero_body(i, _):
        zero_v[pl.ds(i * 16, 16)] = zeros16
        return 0

    lax.fori_loop(0, ZCH // 16, zero_body, 0, unroll=8)

    def ones_body(i, _):
        ones_v[pl.ds(i * 16, 16)] = ones16
        return 0

    lax.fori_loop(0, CHUNK // 16, ones_body, 0, unroll=8)

    for i in range(-(-NZ // NS)):
        j = sid + NS * i

        @pl.when(j < NZ)
        def _(j=j):
            pltpu.sync_copy(zero_v, h_sh.at[pl.ds(j * ZCH, ZCH)])

    plsc.subcore_barrier()

    for b in range(NSETS):
        k = jnp.minimum(wid + NW * b, last)
        pltpu.async_copy(dst_hbm.at[pl.ds(k * CHUNK, CHUNK)], bufs[b], isems[b])

    def round_body(j, _):
        for b in range(NSETS):
            c = wid + NW * (NSETS * j + b)
            pltpu.make_async_copy(
                dst_hbm.at[pl.ds(0, CHUNK)], bufs[b], isems[b]).wait()

            @pl.when(c < NCHUNKS)
            def _(b=b):
                pltpu.async_copy(ones_v, h_sh.at[bufs[b]], csems[b], add=True)

        for b in range(NSETS):
            c = wid + NW * (NSETS * j + b)

            @pl.when(c < NCHUNKS)
            def _(b=b):
                pltpu.make_async_copy(ones_v, h_sh.at[bufs[b]], csems[b]).wait()

            k = jnp.minimum(c + NW * NSETS, last)
            pltpu.async_copy(dst_hbm.at[pl.ds(k * CHUNK, CHUNK)], bufs[b], isems[b])
        return 0

    lax.fori_loop(0, ROUNDS, round_body, 0)
    for b in range(NSETS):
        pltpu.make_async_copy(dst_hbm.at[pl.ds(0, CHUNK)], bufs[b], isems[b]).wait()
    plsc.subcore_barrier()

    for t in range(-(-NZ // NS)):
        j = sid + NS * t

        @pl.when(j < NZ)
        def _(j=j):
            pltpu.sync_copy(h_sh.at[pl.ds(j * ZCH, ZCH)], zero_v)
            pltpu.sync_copy(zero_v, out_hbm.at[pl.ds(cid * NPAD + j * ZCH, ZCH)])


@functools.partial(
    pl.kernel,
    out_type=jax.ShapeDtypeStruct((NC * NPAD,), jnp.float32),
    mesh=_sc_mesh,
    scratch_types=[
        pltpu.VMEM((ZCH,), jnp.float32),
        pltpu.VMEM((CHUNK,), jnp.int32),
        pltpu.VMEM((CHUNK,), jnp.int32),
        pltpu.VMEM((CHUNK,), jnp.int32),
        pltpu.VMEM((CHUNK,), jnp.int32),
        pltpu.VMEM((CHUNK,), jnp.int32),
        pltpu.VMEM((CHUNK,), jnp.int32),
        pltpu.VMEM((CHUNK,), jnp.int32),
        pltpu.VMEM((CHUNK,), jnp.int32),
        pltpu.VMEM((CHUNK,), jnp.float32),
        pltpu.VMEM((CHUNK,), jnp.float32),
        pltpu.VMEM((CHUNK,), jnp.float32),
        pltpu.VMEM((CHUNK,), jnp.float32),
        pltpu.VMEM_SHARED((NPAD,), jnp.float32),
        pltpu.VMEM_SHARED((NPAD,), jnp.float32),
        pltpu.SemaphoreType.DMA,
        pltpu.SemaphoreType.DMA,
        pltpu.SemaphoreType.DMA,
        pltpu.SemaphoreType.DMA,
        pltpu.SemaphoreType.DMA,
        pltpu.SemaphoreType.DMA,
        pltpu.SemaphoreType.DMA,
        pltpu.SemaphoreType.DMA,
        pltpu.SemaphoreType.DMA,
        pltpu.SemaphoreType.DMA,
        pltpu.SemaphoreType.DMA,
        pltpu.SemaphoreType.DMA,
        pltpu.SemaphoreType.DMA,
        pltpu.SemaphoreType.DMA,
        pltpu.SemaphoreType.DMA,
        pltpu.SemaphoreType.DMA,
    ],
)
def _msg_kernel(src_hbm, dst_hbm, d_hbm, out_hbm, zero_v,
                sb0, sb1, sb2, sb3, db0, db1, db2, db3, vb0, vb1, vb2, vb3,
                d_sh, s_sh,
                ss0, ss1, ss2, ss3, sd0, sd1, sd2, sd3,
                sg0, sg1, sg2, sg3, sc0, sc1, sc2, sc3):
    cid = lax.axis_index("c")
    sid = lax.axis_index("s")
    wid = sid * NC + cid
    sbufs = (sb0, sb1, sb2, sb3)
    dbufs = (db0, db1, db2, db3)
    vbufs = (vb0, vb1, vb2, vb3)
    ssems = (ss0, ss1, ss2, ss3)
    dsems = (sd0, sd1, sd2, sd3)
    gsems = (sg0, sg1, sg2, sg3)
    csems = (sc0, sc1, sc2, sc3)
    zeros16 = jnp.zeros((16,), jnp.float32)
    last = NCHUNKS - 1

    def zero_body(i, _):
        zero_v[pl.ds(i * 16, 16)] = zeros16
        return 0

    lax.fori_loop(0, ZCH // 16, zero_body, 0, unroll=8)

    for i in range(-(-NZ // NS)):
        j = sid + NS * i

        @pl.when(j < NZ)
        def _(j=j):
            pltpu.sync_copy(zero_v, s_sh.at[pl.ds(j * ZCH, ZCH)])

    for i in range(-(-NZ // NS)):
        j = sid + NS * i

        @pl.when(j < NZ)
        def _(j=j):
            pltpu.sync_copy(d_hbm.at[pl.ds(j * ZCH, ZCH)], zero_v)
            pltpu.sync_copy(zero_v, d_sh.at[pl.ds(j * ZCH, ZCH)])

    plsc.subcore_barrier()

    for b in range(NSETS):
        k = jnp.minimum(wid + NW * b, last)
        pltpu.async_copy(src_hbm.at[pl.ds(k * CHUNK, CHUNK)], sbufs[b], ssems[b])
        pltpu.async_copy(dst_hbm.at[pl.ds(k * CHUNK, CHUNK)], dbufs[b], dsems[b])

    def round_body(j, _):
        for b in range(NSETS):
            c = wid + NW * (NSETS * j + b)
            pltpu.make_async_copy(
                src_hbm.at[pl.ds(0, CHUNK)], sbufs[b], ssems[b]).wait()

            @pl.when(c < NCHUNKS)
            def _(b=b):
                pltpu.async_copy(d_sh.at[sbufs[b]], vbufs[b], gsems[b])

        for b in range(NSETS):
            c = wid + NW * (NSETS * j + b)
            pltpu.make_async_copy(
                dst_hbm.at[pl.ds(0, CHUNK)], dbufs[b], dsems[b]).wait()

            @pl.when(c < NCHUNKS)
            def _(b=b):
                pltpu.make_async_copy(d_sh.at[sbufs[b]], vbufs[b], gsems[b]).wait()
                pltpu.async_copy(vbufs[b], s_sh.at[dbufs[b]], csems[b], add=True)

        for b in range(NSETS):
            c = wid + NW * (NSETS * j + b)

            @pl.when(c < NCHUNKS)
            def _(b=b):
                pltpu.make_async_copy(vbufs[b], s_sh.at[dbufs[b]], csems[b]).wait()

            k = jnp.minimum(c + NW * NSETS, last)
            pltpu.async_copy(src_hbm.at[pl.ds(k * CHUNK, CHUNK)], sbufs[b], ssems[b])
            pltpu.async_copy(dst_hbm.at[pl.ds(k * CHUNK, CHUNK)], dbufs[b], dsems[b])
        return 0

    lax.fori_loop(0, ROUNDS, round_body, 0)
    for b in range(NSETS):
        pltpu.make_async_copy(src_hbm.at[pl.ds(0, CHUNK)], sbufs[b], ssems[b]).wait()
        pltpu.make_async_copy(dst_hbm.at[pl.ds(0, CHUNK)], dbufs[b], dsems[b]).wait()
    plsc.subcore_barrier()

    for t in range(-(-NZ // NS)):
        j = sid + NS * t

        @pl.when(j < NZ)
        def _(j=j):
            pltpu.sync_copy(s_sh.at[pl.ds(j * ZCH, ZCH)], zero_v)
            pltpu.sync_copy(zero_v, out_hbm.at[pl.ds(cid * NPAD + j * ZCH, ZCH)])


def _rsqrt_body(cnt_ref, d_ref):
    deg = cnt_ref[0, 0, 0, :] + cnt_ref[1, 0, 0, :]
    d_ref[0, 0, :] = lax.rsqrt(deg + 1.0)


_rsqrt_call = pl.pallas_call(
    _rsqrt_body,
    out_shape=jax.ShapeDtypeStruct((NZ, 1, ZCH), jnp.float32),
    grid=(NZ,),
    in_specs=[pl.BlockSpec((NC, 1, 1, ZCH), lambda i: (0, i, 0, 0))],
    out_specs=pl.BlockSpec((1, 1, ZCH), lambda i: (i, 0, 0)),
)


def _final_body(d_ref, s0_ref, s1_ref, wn_ref, wch_ref, wtr_ref, wtx_ref,
                trig_ref, chain_ref, tx_ref, wt_ref, bt_ref, bn_ref,
                wg_ref, bg_ref, out_ref, acc, colsum):
    i = pl.program_id(0)

    @pl.when(i == 0)
    def _():
        acc[...] = jnp.zeros_like(acc)
        colsum[...] = jnp.zeros_like(colsum)

    d = d_ref[0, 0, :]
    g = d * (s0_ref[0, 0, 0, :] + s1_ref[0, 0, 0, :] + d)
    w = wn_ref[...]
    acc[...] += jnp.dot(g.reshape(1, KC), w, preferred_element_type=jnp.float32)
    colsum[...] += jnp.sum(w, axis=0, keepdims=True)

    @pl.when(i == KN - 1)
    def _():
        v = wg_ref[...] * acc[...] + bg_ref[...] * colsum[...]
        trig = jnp.maximum(
            jnp.dot(trig_ref[...], wt_ref[...], preferred_element_type=jnp.float32)
            + bt_ref[...], 0.0)
        out_ref[...] = (
            v
            + chain_ref[...] * wch_ref[...]
            + jnp.dot(trig, wtr_ref[...], preferred_element_type=jnp.float32)
            + jnp.dot(tx_ref[...], wtx_ref[...], preferred_element_type=jnp.float32)
            + bn_ref[...]
        )


_final_call = pl.pallas_call(
    _final_body,
    out_shape=jax.ShapeDtypeStruct((64, 128), jnp.float32),
    grid=(KN,),
    in_specs=[
        pl.BlockSpec((1, 1, KC), lambda i: (i, 0, 0)),
        pl.BlockSpec((1, 1, 1, KC), lambda i: (0, i, 0, 0)),
        pl.BlockSpec((1, 1, 1, KC), lambda i: (1, i, 0, 0)),
        pl.BlockSpec((KC, 128), lambda i: (i, 0)),
        pl.BlockSpec((1, 128), lambda i: (0, 0)),
        pl.BlockSpec((32, 128), lambda i: (0, 0)),
        pl.BlockSpec((8, 128), lambda i: (0, 0)),
        pl.BlockSpec((64, 16), lambda i: (0, 0)),
        pl.BlockSpec((64, 1), lambda i: (0, 0)),
        pl.BlockSpec((64, 8), lambda i: (0, 0)),
        pl.BlockSpec((16, 32), lambda i: (0, 0)),
        pl.BlockSpec((1, 32), lambda i: (0, 0)),
        pl.BlockSpec((1, 128), lambda i: (0, 0)),
        pl.BlockSpec((1, 1), lambda i: (0, 0)),
        pl.BlockSpec((1, 1), lambda i: (0, 0)),
    ],
    out_specs=pl.BlockSpec((64, 128), lambda i: (0, 0)),
    scratch_shapes=[
        pltpu.VMEM((1, 128), jnp.float32),
        pltpu.VMEM((1, 128), jnp.float32),
    ],
)


def kernel(trigger_data, batched_chain, tx_start_time, batched_graphs, edge_index,
           W_gcn, b_gcn, W_t, b_t, W_n, b_n):
    del batched_graphs
    src1d = edge_index[0]
    dst1d = edge_index[1]

    cnt = _hist_kernel(dst1d).reshape(NC, NZ, 1, ZCH)
    d3 = _rsqrt_call(cnt)
    sarr = _msg_kernel(src1d, dst1d, d3.reshape(NPAD)).reshape(NC, NZ, 1, ZCH)

    return _final_call(
        d3, sarr, sarr, W_n,
        W_n[N_NODES:N_NODES + 1],
        W_n[N_NODES + 1:N_NODES + 33],
        W_n[N_NODES + 33:N_NODES + 41],
        trigger_data,
        batched_chain.reshape(64, 1),
        tx_start_time,
        W_t,
        b_t.reshape(1, 32),
        b_n.reshape(1, 128),
        W_gcn,
        b_gcn.reshape(1, 1),
    )
